# baseline (speedup 1.0000x reference)
import functools
import math

import jax
import jax.numpy as jnp
from jax import lax
from jax.experimental import pallas as pl
from jax.experimental.pallas import tpu as pltpu
from jax.experimental.pallas import tpu_sc as plsc

D = 64
B = 16384
SCALE = 1.0 / math.sqrt(4.0)

NC = 2
NS = 16
NW = NC * NS
BPW = B // NW
CH = 128
NCHUNK = BPW // CH


def _adj_body(d_ref, w_ref, u_ref, i_ref, c_ref, e0_ref, o_ref):
    acc = jnp.dot(d_ref[...], w_ref[...], preferred_element_type=jnp.float32)
    mu = (u_ref[...] == 0).astype(jnp.float32)
    mi = (i_ref[...] == 0).astype(jnp.float32)
    mc = (c_ref[...] == 0).astype(jnp.float32)
    corr = (mu * e0_ref[0:1, :] + mi * e0_ref[1:2, :] + mc * e0_ref[2:3, :])
    o_ref[...] = (acc - corr) * SCALE


def _adjustment(dense_0, W_dense, u2, i2, c2, e0):
    return pl.pallas_call(
        _adj_body,
        out_shape=jax.ShapeDtypeStruct((B, D), jnp.float32),
    )(dense_0, W_dense, u2, i2, c2, e0)


def _sc_body(uid_hbm, iid_hbm, cid_hbm, adj_hbm, eu_hbm, ei_hbm, ec_hbm,
             out_hbm, idxu_v, idxi_v, idxc_v, ru_v, ri_v, rc_v, adj_v, sem):
    wid = lax.axis_index("s") * NC + lax.axis_index("c")
    base = wid * BPW

    def chunk(ci, carry):
        off = base + ci * CH
        pltpu.sync_copy(uid_hbm.at[pl.ds(off, CH)], idxu_v)
        pltpu.sync_copy(iid_hbm.at[pl.ds(off, CH)], idxi_v)
        pltpu.sync_copy(cid_hbm.at[pl.ds(off, CH)], idxc_v)
        cu = pltpu.async_copy(eu_hbm.at[idxu_v], ru_v, sem)
        cit = pltpu.async_copy(ei_hbm.at[idxi_v], ri_v, sem)
        cc = pltpu.async_copy(ec_hbm.at[idxc_v], rc_v, sem)
        ca = pltpu.async_copy(adj_hbm.at[pl.ds(off, CH)], adj_v, sem)
        cu.wait()
        cit.wait()
        cc.wait()
        ca.wait()

        def row(r, rcarry):
            for c4 in range(D // 16):
                s = pl.ds(c4 * 16, 16)
                ru_v[r, s] = (ru_v[r, s] + ri_v[r, s] + rc_v[r, s]) * SCALE \
                    + adj_v[r, s]
            return rcarry

        lax.fori_loop(0, CH, row, 0)
        pltpu.sync_copy(ru_v, out_hbm.at[pl.ds(off, CH)])
        return carry

    lax.fori_loop(0, NCHUNK, chunk, 0)


_sc_call = functools.partial(
    pl.kernel,
    out_type=jax.ShapeDtypeStruct((B, D), jnp.float32),
    mesh=plsc.VectorSubcoreMesh(core_axis_name="c", subcore_axis_name="s"),
    scratch_types=[
        pltpu.VMEM((CH,), jnp.int32),
        pltpu.VMEM((CH,), jnp.int32),
        pltpu.VMEM((CH,), jnp.int32),
        pltpu.VMEM((CH, D), jnp.float32),
        pltpu.VMEM((CH, D), jnp.float32),
        pltpu.VMEM((CH, D), jnp.float32),
        pltpu.VMEM((CH, D), jnp.float32),
        pltpu.SemaphoreType.DMA,
    ],
    compiler_params=pltpu.CompilerParams(use_tc_tiling_on_sc=False),
)(_sc_body)


def kernel(user_id, item_id, category, dense_0, E_user, E_item, E_cat,
           W_dense):
    u = user_id.astype(jnp.int32)
    i = item_id.astype(jnp.int32)
    c = category.astype(jnp.int32)
    e0 = jnp.stack([E_user[0], E_item[0], E_cat[0]], axis=0)
    adj = _adjustment(dense_0, W_dense, u[:, None], i[:, None], c[:, None],
                      e0)
    return _sc_call(u, i, c, adj, E_user, E_item, E_cat)


# speedup vs baseline: 1.5391x; 1.5391x over previous
import functools
import math

import jax
import jax.numpy as jnp
from jax import lax
from jax.experimental import pallas as pl
from jax.experimental.pallas import tpu as pltpu
from jax.experimental.pallas import tpu_sc as plsc

D = 64
B = 16384
SCALE = 1.0 / math.sqrt(4.0)

NC = 2
NS = 16
NW = NC * NS
BPW = B // NW
CH = 128
NCHUNK = BPW // CH


def _adj_body(d_ref, w_ref, u_ref, i_ref, c_ref, e0_ref, o_ref):
    acc = jnp.dot(d_ref[...], w_ref[...], preferred_element_type=jnp.float32)
    mu = (u_ref[...] == 0).astype(jnp.float32)
    mi = (i_ref[...] == 0).astype(jnp.float32)
    mc = (c_ref[...] == 0).astype(jnp.float32)
    corr = (mu * e0_ref[0:1, :] + mi * e0_ref[1:2, :] + mc * e0_ref[2:3, :])
    o_ref[...] = (acc - corr) * SCALE


def _adjustment(dense_0, W_dense, u2, i2, c2, e0):
    return pl.pallas_call(
        _adj_body,
        out_shape=jax.ShapeDtypeStruct((B, D), jnp.float32),
    )(dense_0, W_dense, u2, i2, c2, e0)


def _sc_body(uid_hbm, iid_hbm, cid_hbm, adj_hbm, eu_hbm, ei_hbm, ec_hbm,
             out_hbm, idxu_s, idxi_s, idxc_s, ru_v, ri_v, rc_v,
             adj_v, sem, adj_sem):
    wid = lax.axis_index("s") * NC + lax.axis_index("c")
    base = wid * BPW

    def chunk(ci, carry):
        off = base + ci * CH
        pltpu.sync_copy(uid_hbm.at[pl.ds(off, CH)], idxu_s)
        pltpu.sync_copy(iid_hbm.at[pl.ds(off, CH)], idxi_s)
        pltpu.sync_copy(cid_hbm.at[pl.ds(off, CH)], idxc_s)
        ca = pltpu.async_copy(adj_hbm.at[pl.ds(off, CH)], adj_v, adj_sem)

        def issue(g, rcarry):
            vu = idxu_s[pl.ds(g * 16, 16)]
            vi = idxi_s[pl.ds(g * 16, 16)]
            vc = idxc_s[pl.ds(g * 16, 16)]
            for l in range(16):
                r = g * 16 + l
                pltpu.async_copy(eu_hbm.at[pl.ds(vu[l], 1)],
                                 ru_v.at[pl.ds(r, 1)], sem)
                pltpu.async_copy(ei_hbm.at[pl.ds(vi[l], 1)],
                                 ri_v.at[pl.ds(r, 1)], sem)
                pltpu.async_copy(ec_hbm.at[pl.ds(vc[l], 1)],
                                 rc_v.at[pl.ds(r, 1)], sem)
            return rcarry

        lax.fori_loop(0, CH // 16, issue, 0)
        pltpu.make_async_copy(eu_hbm.at[pl.ds(0, CH)], ru_v, sem).wait()
        pltpu.make_async_copy(ei_hbm.at[pl.ds(0, CH)], ri_v, sem).wait()
        pltpu.make_async_copy(ec_hbm.at[pl.ds(0, CH)], rc_v, sem).wait()
        ca.wait()

        def row(r, rcarry):
            for c4 in range(D // 16):
                s = pl.ds(c4 * 16, 16)
                ru_v[r, s] = (ru_v[r, s] + ri_v[r, s] + rc_v[r, s]) * SCALE \
                    + adj_v[r, s]
            return rcarry

        lax.fori_loop(0, CH, row, 0)
        pltpu.sync_copy(ru_v, out_hbm.at[pl.ds(off, CH)])
        return carry

    lax.fori_loop(0, NCHUNK, chunk, 0)


_sc_call = functools.partial(
    pl.kernel,
    out_type=jax.ShapeDtypeStruct((B, D), jnp.float32),
    mesh=plsc.VectorSubcoreMesh(core_axis_name="c", subcore_axis_name="s"),
    scratch_types=[
        pltpu.VMEM((CH,), jnp.int32),
        pltpu.VMEM((CH,), jnp.int32),
        pltpu.VMEM((CH,), jnp.int32),
        pltpu.VMEM((CH, D), jnp.float32),
        pltpu.VMEM((CH, D), jnp.float32),
        pltpu.VMEM((CH, D), jnp.float32),
        pltpu.VMEM((CH, D), jnp.float32),
        pltpu.SemaphoreType.DMA,
        pltpu.SemaphoreType.DMA,
    ],
)(_sc_body)


def kernel(user_id, item_id, category, dense_0, E_user, E_item, E_cat,
           W_dense):
    u = user_id.astype(jnp.int32)
    i = item_id.astype(jnp.int32)
    c = category.astype(jnp.int32)
    e0 = jnp.stack([E_user[0], E_item[0], E_cat[0]], axis=0)
    adj = _adjustment(dense_0, W_dense, u[:, None], i[:, None], c[:, None],
                      e0)
    return _sc_call(u, i, c, adj, E_user, E_item, E_cat)
